# baseline (speedup 1.0000x reference)
import functools

import jax
import jax.numpy as jnp
from jax import lax
from jax.experimental import pallas as pl
from jax.experimental.pallas import tpu as pltpu
from jax.experimental.pallas import tpu_sc as plsc

BATCH = 4096
SEQ = 200
EMB = 64
VOCAB = 1000000
LANES = 16

_NC = 2
_NS = 16
_NW = _NC * _NS
_LBLK = 8
_BBLK = 128
_NUNITS = (SEQ // _LBLK) * (BATCH // _BBLK)
_UPW = _NUNITS // _NW
_BCOLS = BATCH // _BBLK


def _build():
    mesh = plsc.VectorSubcoreMesh(core_axis_name="c", subcore_axis_name="s")

    @functools.partial(
        pl.kernel,
        out_type=jax.ShapeDtypeStruct((SEQ, EMB, BATCH), jnp.float32),
        mesh=mesh,
        compiler_params=pltpu.CompilerParams(needs_layout_passes=False),
        scratch_types=[
            pltpu.VMEM((_LBLK, _BBLK), jnp.int32),
            pltpu.VMEM((_BBLK,), jnp.int32),
            pltpu.VMEM((_BBLK,), jnp.int32),
            pltpu.VMEM((_BBLK, 128), jnp.float32),
            pltpu.VMEM((EMB, _BBLK), jnp.float32),
            pltpu.VMEM((EMB, 256), jnp.float32),
            pltpu.SemaphoreType.DMA,
        ],
    )
    def k(xT_hbm, tab_hbm, posT_hbm, out_hbm,
          xv, r2_v, pb_v, gbuf, otile, pos_v, sem_g):
        wid = lax.axis_index("s") * _NC + lax.axis_index("c")
        pltpu.sync_copy(posT_hbm.at[pl.ds(0, EMB), pl.ds(0, 256)], pos_v)
        row_ids = [lax.iota(jnp.int32, LANES) + j * LANES
                   for j in range(_BBLK // LANES)]

        def unit_body(t, c0):
            u = wid * _UPW + t
            l0 = (u // _BCOLS) * _LBLK
            b0 = (u % _BCOLS) * _BBLK
            pltpu.sync_copy(xT_hbm.at[pl.ds(l0, _LBLK), pl.ds(b0, _BBLK)], xv)

            def l_body(l_off, c1):
                for j in range(_BBLK // LANES):
                    sl = pl.ds(j * LANES, LANES)
                    idx = xv[l_off, sl]
                    r2_v[sl] = lax.shift_right_logical(idx, 1)
                    pb_v[sl] = lax.shift_left(jnp.bitwise_and(idx, 1), 6)
                pltpu.async_copy(tab_hbm.at[r2_v], gbuf, sem_g).wait()

                lvec = jnp.broadcast_to(l0 + l_off, (LANES,)).astype(jnp.int32)

                def e_body(e, c2):
                    ev = jnp.broadcast_to(e, (LANES,)).astype(jnp.int32)
                    pvec = plsc.load_gather(pos_v, [ev, lvec])
                    for j in range(_BBLK // LANES):
                        sl = pl.ds(j * LANES, LANES)
                        col = pb_v[sl] + e
                        val = plsc.load_gather(gbuf, [row_ids[j], col])
                        otile[e, sl] = val + pvec
                    return c2

                lax.fori_loop(0, EMB, e_body, 0)
                pltpu.sync_copy(
                    otile, out_hbm.at[l0 + l_off, pl.ds(0, EMB), pl.ds(b0, _BBLK)])
                return c1

            lax.fori_loop(0, _LBLK, l_body, 0)
            return c0

        lax.fori_loop(0, _UPW, unit_body, 0)

    return k


_KERNEL = _build()


def kernel(x, table, pos_encoding):
    xT = x.astype(jnp.int32).T
    tab128 = table.reshape(VOCAB // 2, 2 * EMB)
    posT = pos_encoding.T
    out_p = _KERNEL(xT, tab128, posT)
    return jnp.transpose(out_p, (2, 0, 1))


# speedup vs baseline: 1.3661x; 1.3661x over previous
import functools

import jax
import jax.numpy as jnp
from jax import lax
from jax.experimental import pallas as pl
from jax.experimental.pallas import tpu as pltpu
from jax.experimental.pallas import tpu_sc as plsc

BATCH = 4096
SEQ = 200
EMB = 64
VOCAB = 1000000
LANES = 16

_NC = 2
_NS = 16
_NW = _NC * _NS
_LBLK = 8
_BBLK = 128
_NJ = _BBLK // LANES
_NUNITS = (SEQ // _LBLK) * (BATCH // _BBLK)
_UPW = _NUNITS // _NW
_BCOLS = BATCH // _BBLK


def _build():
    mesh = plsc.VectorSubcoreMesh(core_axis_name="c", subcore_axis_name="s")

    @functools.partial(
        pl.kernel,
        out_type=jax.ShapeDtypeStruct((SEQ, EMB, BATCH), jnp.float32),
        mesh=mesh,
        compiler_params=pltpu.CompilerParams(needs_layout_passes=False),
        scratch_types=[
            pltpu.VMEM((_LBLK, _BBLK), jnp.int32),
            pltpu.VMEM((_BBLK,), jnp.int32),
            pltpu.VMEM((_BBLK,), jnp.int32),
            pltpu.VMEM((_BBLK,), jnp.int32),
            pltpu.VMEM((_BBLK,), jnp.int32),
            pltpu.VMEM((_BBLK, 128), jnp.float32),
            pltpu.VMEM((_BBLK, 128), jnp.float32),
            pltpu.VMEM((EMB, _BBLK), jnp.float32),
            pltpu.VMEM((EMB, _BBLK), jnp.float32),
            pltpu.VMEM((EMB, 256), jnp.float32),
            pltpu.SemaphoreType.DMA,
            pltpu.SemaphoreType.DMA,
            pltpu.SemaphoreType.DMA,
            pltpu.SemaphoreType.DMA,
        ],
    )
    def k(xT_hbm, tab_hbm, posT_hbm, out_hbm,
          xv, r2_0, r2_1, pb_0, pb_1, gb_0, gb_1, ot_0, ot_1, pos_v,
          sg_0, sg_1, so_0, so_1):
        wid = lax.axis_index("s") * _NC + lax.axis_index("c")
        pltpu.sync_copy(posT_hbm.at[pl.ds(0, EMB), pl.ds(0, 256)], pos_v)
        row_ids = [lax.iota(jnp.int32, LANES) + j * LANES for j in range(_NJ)]

        def prep(l_off, r2_v, pb_v):
            for j in range(_NJ):
                sl = pl.ds(j * LANES, LANES)
                idx = xv[l_off, sl]
                r2_v[sl] = lax.shift_right_logical(idx, 1)
                pb_v[sl] = lax.shift_left(jnp.bitwise_and(idx, 1), 6)

        def fire_gather(r2_v, gb, sg):
            return pltpu.async_copy(tab_hbm.at[r2_v], gb, sg)

        def wait_gather(r2_v, gb, sg):
            pltpu.make_async_copy(tab_hbm.at[r2_v], gb, sg).wait()

        def compute(l, pb_v, gb, ot):
            lvec = jnp.broadcast_to(l, (LANES,)).astype(jnp.int32)
            pbs = [pb_v[pl.ds(j * LANES, LANES)] for j in range(_NJ)]
            for e in range(EMB):
                ev = jnp.full((LANES,), e, jnp.int32)
                pvec = plsc.load_gather(pos_v, [ev, lvec])
                for j in range(_NJ):
                    val = plsc.load_gather(gb, [row_ids[j], pbs[j] + e])
                    ot[e, pl.ds(j * LANES, LANES)] = val + pvec

        def out_ref(l, b0):
            return out_hbm.at[l, pl.ds(0, EMB), pl.ds(b0, _BBLK)]

        def unit_body(t, c0):
            u = wid * _UPW + t
            l0 = (u // _BCOLS) * _LBLK
            b0 = (u % _BCOLS) * _BBLK
            pltpu.sync_copy(xT_hbm.at[pl.ds(l0, _LBLK), pl.ds(b0, _BBLK)], xv)
            prep(0, r2_0, pb_0)
            fire_gather(r2_0, gb_0, sg_0)

            def pair_body(p, c1):
                le = 2 * p
                lo = 2 * p + 1
                not_first = jnp.logical_or(t > 0, p > 0)

                prep(lo, r2_1, pb_1)
                fire_gather(r2_1, gb_1, sg_1)
                wait_gather(r2_0, gb_0, sg_0)

                @pl.when(not_first)
                def _():
                    pltpu.make_async_copy(ot_0, out_ref(l0 + le, b0), so_0).wait()

                compute(l0 + le, pb_0, gb_0, ot_0)
                pltpu.async_copy(ot_0, out_ref(l0 + le, b0), so_0)

                @pl.when(p < 3)
                def _():
                    prep(le + 2, r2_0, pb_0)
                    fire_gather(r2_0, gb_0, sg_0)

                wait_gather(r2_1, gb_1, sg_1)

                @pl.when(not_first)
                def _():
                    pltpu.make_async_copy(ot_1, out_ref(l0 + lo, b0), so_1).wait()

                compute(l0 + lo, pb_1, gb_1, ot_1)
                pltpu.async_copy(ot_1, out_ref(l0 + lo, b0), so_1)
                return c1

            lax.fori_loop(0, _LBLK // 2, pair_body, 0)
            return c0

        lax.fori_loop(0, _UPW, unit_body, 0)
        pltpu.make_async_copy(ot_0, out_ref(SEQ - 2, 0), so_0).wait()
        pltpu.make_async_copy(ot_1, out_ref(SEQ - 1, 0), so_1).wait()

    return k


_KERNEL = _build()


def kernel(x, table, pos_encoding):
    xT = x.astype(jnp.int32).T
    tab128 = table.reshape(VOCAB // 2, 2 * EMB)
    posT = pos_encoding.T
    out_p = _KERNEL(xT, tab128, posT)
    return jnp.transpose(out_p, (2, 0, 1))


# speedup vs baseline: 1.5141x; 1.1083x over previous
import functools

import jax
import jax.numpy as jnp
from jax import lax
from jax.experimental import pallas as pl
from jax.experimental.pallas import tpu as pltpu
from jax.experimental.pallas import tpu_sc as plsc

BATCH = 4096
SEQ = 200
EMB = 64
VOCAB = 1000000
LANES = 16

_NC = 2
_NS = 16
_NW = _NC * _NS
_LBLK = 8
_BBLK = 128
_NJ = _BBLK // LANES
_NUNITS = (SEQ // _LBLK) * (BATCH // _BBLK)
_UPW = _NUNITS // _NW
_BCOLS = BATCH // _BBLK
_ITER = _UPW * _LBLK
_DEPTH = 4


def _build():
    mesh = plsc.VectorSubcoreMesh(core_axis_name="c", subcore_axis_name="s")

    @functools.partial(
        pl.kernel,
        out_type=jax.ShapeDtypeStruct((SEQ, EMB, BATCH), jnp.float32),
        mesh=mesh,
        compiler_params=pltpu.CompilerParams(needs_layout_passes=False),
        scratch_types=[
            pltpu.VMEM((2, _LBLK, _BBLK), jnp.int32),
            [pltpu.VMEM((_BBLK,), jnp.int32) for _ in range(_DEPTH)],
            [pltpu.VMEM((_BBLK,), jnp.int32) for _ in range(_DEPTH)],
            [pltpu.VMEM((_BBLK, 128), jnp.float32) for _ in range(_DEPTH)],
            [pltpu.VMEM((EMB, _BBLK), jnp.float32) for _ in range(2)],
            pltpu.VMEM((EMB, 256), jnp.float32),
            [pltpu.SemaphoreType.DMA for _ in range(_DEPTH)],
            [pltpu.SemaphoreType.DMA for _ in range(2)],
        ],
    )
    def k(xT_hbm, tab_hbm, posT_hbm, out_hbm,
          xv, r2s, pbs_v, gbs, ots, pos_v, sgs, sos):
        wid = lax.axis_index("s") * _NC + lax.axis_index("c")
        pltpu.sync_copy(posT_hbm.at[pl.ds(0, EMB), pl.ds(0, 256)], pos_v)
        row_ids = [lax.iota(jnp.int32, LANES) + j * LANES for j in range(_NJ)]

        def coords(i):
            gu = wid * _UPW + i // _LBLK
            l = (gu // _BCOLS) * _LBLK + i % _LBLK
            b0 = (gu % _BCOLS) * _BBLK
            return l, b0

        def load_xv(i):
            gu = wid * _UPW + i // _LBLK
            l0 = (gu // _BCOLS) * _LBLK
            b0 = (gu % _BCOLS) * _BBLK
            pltpu.sync_copy(
                xT_hbm.at[pl.ds(l0, _LBLK), pl.ds(b0, _BBLK)],
                xv.at[(i // _LBLK) % 2])

        def prep_fire(i, b):
            sel = (i // _LBLK) % 2
            l_off = i % _LBLK
            for j in range(_NJ):
                sl = pl.ds(j * LANES, LANES)
                idx = xv[sel, l_off, sl]
                r2s[b][sl] = lax.shift_right_logical(idx, 1)
                pbs_v[b][sl] = lax.shift_left(jnp.bitwise_and(idx, 1), 6)
            pltpu.async_copy(tab_hbm.at[r2s[b]], gbs[b], sgs[b])

        def out_ref(l, b0):
            return out_hbm.at[l, pl.ds(0, EMB), pl.ds(b0, _BBLK)]

        def compute(i, b, ob):
            l, b0 = coords(i)
            lvec = jnp.broadcast_to(l, (LANES,)).astype(jnp.int32)
            pb = [pbs_v[b][pl.ds(j * LANES, LANES)] for j in range(_NJ)]

            def e_body(e, c):
                ev = jnp.broadcast_to(e, (LANES,)).astype(jnp.int32)
                pvec = plsc.load_gather(pos_v, [ev, lvec])
                for j in range(_NJ):
                    val = plsc.load_gather(gbs[b], [row_ids[j], pb[j] + e])
                    ots[ob][e, pl.ds(j * LANES, LANES)] = val + pvec
                return c

            lax.fori_loop(0, EMB, e_body, 0, unroll=8)
            pltpu.async_copy(ots[ob], out_ref(l, b0), sos[ob])

        load_xv(0)
        for w in range(_DEPTH - 1):
            prep_fire(w, w)

        def q_body(q, c0):
            for c in range(_DEPTH):
                i = _DEPTH * q + c
                ip = i + (_DEPTH - 1)
                bp = (c + _DEPTH - 1) % _DEPTH

                @pl.when(jnp.logical_and(ip < _ITER, ip % _LBLK == 0))
                def _():
                    load_xv(ip)

                @pl.when(ip < _ITER)
                def _():
                    prep_fire(ip, bp)

                pltpu.make_async_copy(tab_hbm.at[r2s[c]], gbs[c], sgs[c]).wait()

                @pl.when(i >= 2)
                def _():
                    l2, b02 = coords(i)
                    pltpu.make_async_copy(
                        ots[c % 2], out_ref(l2, b02), sos[c % 2]).wait()

                compute(i, c, c % 2)
            return c0

        lax.fori_loop(0, _ITER // _DEPTH, q_body, 0)
        pltpu.make_async_copy(ots[0], out_ref(0, 0), sos[0]).wait()
        pltpu.make_async_copy(ots[1], out_ref(0, 0), sos[1]).wait()

    return k


_KERNEL = _build()


def kernel(x, table, pos_encoding):
    xT = x.astype(jnp.int32).T
    tab128 = table.reshape(VOCAB // 2, 2 * EMB)
    posT = pos_encoding.T
    out_p = _KERNEL(xT, tab128, posT)
    return jnp.transpose(out_p, (2, 0, 1))


# speedup vs baseline: 1.7482x; 1.1546x over previous
import functools

import jax
import jax.numpy as jnp
from jax import lax
from jax.experimental import pallas as pl
from jax.experimental.pallas import tpu as pltpu
from jax.experimental.pallas import tpu_sc as plsc

BATCH = 4096
SEQ = 200
EMB = 64
VOCAB = 1000000
LANES = 16

_NC = 2
_NS = 16
_NW = _NC * _NS
_LBLK = 8
_BBLK = 128
_NJ = _BBLK // LANES
_NE = EMB // LANES
_NUNITS = (SEQ // _LBLK) * (BATCH // _BBLK)
_UPW = _NUNITS // _NW
_BCOLS = BATCH // _BBLK
_ITER = _UPW * _LBLK
_DEPTH = 4


def _build():
    mesh = plsc.VectorSubcoreMesh(core_axis_name="c", subcore_axis_name="s")

    @functools.partial(
        pl.kernel,
        out_type=jax.ShapeDtypeStruct((SEQ, BATCH // 2, 2 * EMB), jnp.float32),
        mesh=mesh,
        compiler_params=pltpu.CompilerParams(needs_layout_passes=False),
        scratch_types=[
            pltpu.VMEM((2, _LBLK, _BBLK), jnp.int32),
            [pltpu.VMEM((_BBLK,), jnp.int32) for _ in range(_DEPTH)],
            [pltpu.VMEM((_BBLK,), jnp.int32) for _ in range(_DEPTH)],
            [pltpu.VMEM((_BBLK, 128), jnp.float32) for _ in range(_DEPTH)],
            [pltpu.VMEM((_BBLK // 2, 128), jnp.float32) for _ in range(2)],
            pltpu.VMEM((EMB, 256), jnp.float32),
            [pltpu.SemaphoreType.DMA for _ in range(_DEPTH)],
            [pltpu.SemaphoreType.DMA for _ in range(2)],
        ],
    )
    def k(xT_hbm, tab_hbm, posT_hbm, out_hbm,
          xv, r2s, pbs, gbs, ots, pos_v, sgs, sos):
        wid = lax.axis_index("s") * _NC + lax.axis_index("c")
        pltpu.sync_copy(posT_hbm.at[pl.ds(0, EMB), pl.ds(0, 256)], pos_v)

        def coords(i):
            gu = wid * _UPW + i // _LBLK
            l = (gu // _BCOLS) * _LBLK + i % _LBLK
            b0 = (gu % _BCOLS) * _BBLK
            return l, b0

        def load_xv(i):
            gu = wid * _UPW + i // _LBLK
            l0 = (gu // _BCOLS) * _LBLK
            b0 = (gu % _BCOLS) * _BBLK
            pltpu.sync_copy(
                xT_hbm.at[pl.ds(l0, _LBLK), pl.ds(b0, _BBLK)],
                xv.at[(i // _LBLK) % 2])

        def prep_fire(i, b):
            sel = (i // _LBLK) % 2
            l_off = i % _LBLK
            for j in range(_NJ):
                sl = pl.ds(j * LANES, LANES)
                idx = xv[sel, l_off, sl]
                r2s[b][sl] = lax.shift_right_logical(idx, 1)
                pbs[b][sl] = lax.shift_left(jnp.bitwise_and(idx, 1), 6)
            pltpu.async_copy(tab_hbm.at[r2s[b]], gbs[b], sgs[b])

        def out_ref(l, b0):
            b0h = pl.multiple_of(b0 // 2, _BBLK // 2)
            return out_hbm.at[l, pl.ds(b0h, _BBLK // 2), pl.ds(0, 128)]

        ecols = [lax.iota(jnp.int32, LANES) + eg * LANES for eg in range(_NE)]

        def compute(i, b, ob):
            l, b0 = coords(i)
            lvec = jnp.broadcast_to(l, (LANES,)).astype(jnp.int32)
            pvecs = [plsc.load_gather(pos_v, [ecols[eg], lvec])
                     for eg in range(_NE)]

            def b_body(bb, c):
                bsplat = jnp.broadcast_to(bb, (LANES,)).astype(jnp.int32)
                pbspl = plsc.load_gather(pbs[b], [bsplat])
                half = lax.shift_left(jnp.bitwise_and(bb, 1), 6)
                row2 = bb // 2
                for eg in range(_NE):
                    val = plsc.load_gather(gbs[b], [bsplat, pbspl + ecols[eg]])
                    ots[ob][row2, pl.ds(half + eg * LANES, LANES)] = (
                        val + pvecs[eg])
                return c

            lax.fori_loop(0, _BBLK, b_body, 0, unroll=8)
            pltpu.async_copy(ots[ob], out_ref(l, b0), sos[ob])

        load_xv(0)
        for w in range(_DEPTH - 1):
            prep_fire(w, w)

        def q_body(q, c0):
            for c in range(_DEPTH):
                i = _DEPTH * q + c
                ip = i + (_DEPTH - 1)
                bp = (c + _DEPTH - 1) % _DEPTH

                @pl.when(jnp.logical_and(ip < _ITER, ip % _LBLK == 0))
                def _():
                    load_xv(ip)

                @pl.when(ip < _ITER)
                def _():
                    prep_fire(ip, bp)

                pltpu.make_async_copy(tab_hbm.at[r2s[c]], gbs[c], sgs[c]).wait()

                @pl.when(i >= 2)
                def _():
                    l2, b02 = coords(i)
                    pltpu.make_async_copy(
                        ots[c % 2], out_ref(l2, b02), sos[c % 2]).wait()

                compute(i, c, c % 2)
            return c0

        lax.fori_loop(0, _ITER // _DEPTH, q_body, 0)
        pltpu.make_async_copy(ots[0], out_ref(0, 0), sos[0]).wait()
        pltpu.make_async_copy(ots[1], out_ref(0, 0), sos[1]).wait()

    return k


_KERNEL = _build()


def kernel(x, table, pos_encoding):
    xT = x.astype(jnp.int32).T
    tab128 = table.reshape(VOCAB // 2, 2 * EMB)
    posT = pos_encoding.T
    out_p = _KERNEL(xT, tab128, posT)
    out_tok = out_p.reshape(SEQ, BATCH, EMB)
    return jnp.transpose(out_tok, (1, 0, 2))


# speedup vs baseline: 2.0078x; 1.1485x over previous
import functools

import jax
import jax.numpy as jnp
from jax import lax
from jax.experimental import pallas as pl
from jax.experimental.pallas import tpu as pltpu
from jax.experimental.pallas import tpu_sc as plsc

BATCH = 4096
SEQ = 200
EMB = 64
VOCAB = 1000000
LANES = 16

_NC = 2
_NS = 16
_NW = _NC * _NS
_LBLK = 8
_BBLK = 128
_NE = EMB // LANES
_NJ = _BBLK // LANES
_NUNITS = (SEQ // _LBLK) * (BATCH // _BBLK)
_UPW = _NUNITS // _NW
_BCOLS = BATCH // _BBLK
_ITER = _UPW * _LBLK
_DEPTH = 4


def _build():
    mesh = plsc.VectorSubcoreMesh(core_axis_name="c", subcore_axis_name="s")

    @functools.partial(
        pl.kernel,
        out_type=jax.ShapeDtypeStruct((BATCH, SEQ * EMB), jnp.float32),
        mesh=mesh,
        compiler_params=pltpu.CompilerParams(needs_layout_passes=False),
        scratch_types=[
            pltpu.VMEM((2, _LBLK, _BBLK), jnp.int32),
            [pltpu.VMEM((_BBLK,), jnp.int32) for _ in range(_DEPTH)],
            [pltpu.VMEM((_BBLK,), jnp.int32) for _ in range(_DEPTH)],
            [pltpu.VMEM((_BBLK, 128), jnp.float32) for _ in range(_DEPTH)],
            [pltpu.VMEM((_BBLK, 128), jnp.float32) for _ in range(2)],
            pltpu.VMEM((EMB, 256), jnp.float32),
            [pltpu.SemaphoreType.DMA for _ in range(_DEPTH)],
            [pltpu.SemaphoreType.DMA for _ in range(2)],
        ],
    )
    def k(xT_hbm, tab_hbm, posT_hbm, out_hbm,
          xv, r2s, pbs, gbs, ots, pos_v, sgs, sos):
        wid = lax.axis_index("s") * _NC + lax.axis_index("c")
        pltpu.sync_copy(posT_hbm.at[pl.ds(0, EMB), pl.ds(0, 256)], pos_v)
        ecols = [lax.iota(jnp.int32, LANES) + eg * LANES for eg in range(_NE)]

        def coords(i):
            gu = wid * _UPW + i // _LBLK
            l = (gu // _BCOLS) * _LBLK + i % _LBLK
            b0 = (gu % _BCOLS) * _BBLK
            return l, b0

        def load_xv(i):
            gu = wid * _UPW + i // _LBLK
            l0 = (gu // _BCOLS) * _LBLK
            b0 = (gu % _BCOLS) * _BBLK
            pltpu.sync_copy(
                xT_hbm.at[pl.ds(l0, _LBLK), pl.ds(b0, _BBLK)],
                xv.at[(i // _LBLK) % 2])

        def prep_fire(i, b):
            sel = (i // _LBLK) % 2
            l_off = i % _LBLK
            for j in range(_NJ):
                sl = pl.ds(j * LANES, LANES)
                idx = xv[sel, l_off, sl]
                r2s[b][sl] = lax.shift_right_logical(idx, 1)
                pbs[b][sl] = lax.shift_left(jnp.bitwise_and(idx, 1), 6)
            pltpu.async_copy(tab_hbm.at[r2s[b]], gbs[b], sgs[b])

        def out_ref(i):
            l, b0 = coords(i)
            col = pl.multiple_of((l // 2) * 2 * EMB, 2 * EMB)
            return out_hbm.at[pl.ds(b0, _BBLK), pl.ds(col, 2 * EMB)]

        def compute(i, b, ob, half):
            l, _ = coords(i)
            lvec = jnp.broadcast_to(l, (LANES,)).astype(jnp.int32)
            pvecs = [plsc.load_gather(pos_v, [ecols[eg], lvec])
                     for eg in range(_NE)]

            def b_body(bb, c):
                bsplat = jnp.broadcast_to(bb, (LANES,)).astype(jnp.int32)
                pbspl = plsc.load_gather(pbs[b], [bsplat])
                for eg in range(_NE):
                    val = plsc.load_gather(gbs[b], [bsplat, pbspl + ecols[eg]])
                    ots[ob][bb, pl.ds(half * EMB + eg * LANES, LANES)] = (
                        val + pvecs[eg])
                return c

            lax.fori_loop(0, _BBLK, b_body, 0, unroll=8)

        load_xv(0)
        for w in range(_DEPTH - 1):
            prep_fire(w, w)

        def q_body(qq, c0):
            for c in range(_LBLK):
                i = _LBLK * qq + c
                ip = i + (_DEPTH - 1)
                bp = (c + _DEPTH - 1) % _DEPTH
                ob = (c // 2) % 2

                if c == _LBLK - (_DEPTH - 1):
                    @pl.when(ip < _ITER)
                    def _():
                        load_xv(ip)

                @pl.when(ip < _ITER)
                def _():
                    prep_fire(ip, bp)

                pltpu.make_async_copy(tab_hbm.at[r2s[c % _DEPTH]],
                                      gbs[c % _DEPTH], sgs[c % _DEPTH]).wait()

                if c % 2 == 0:
                    @pl.when(i >= 4)
                    def _():
                        pltpu.make_async_copy(ots[ob], out_ref(i), sos[ob]).wait()

                compute(i, c % _DEPTH, ob, c % 2)

                if c % 2 == 1:
                    pltpu.async_copy(ots[ob], out_ref(i), sos[ob])
            return c0

        lax.fori_loop(0, _ITER // _LBLK, q_body, 0)
        pltpu.make_async_copy(ots[0], out_ref(0), sos[0]).wait()
        pltpu.make_async_copy(ots[1], out_ref(2), sos[1]).wait()

    return k


_KERNEL = _build()


def kernel(x, table, pos_encoding):
    xT = x.astype(jnp.int32).T
    tab128 = table.reshape(VOCAB // 2, 2 * EMB)
    posT = pos_encoding.T
    out_p = _KERNEL(xT, tab128, posT)
    return out_p.reshape(BATCH, SEQ, EMB)
